# Initial kernel scaffold; baseline (speedup 1.0000x reference)
import jax
import jax.numpy as jnp
from jax.experimental import pallas as pl


def kernel(x, edge_index, edge_attr, batch, mask, W0, b0, g0, be0, W1, b1, g1, be1, Wf1, bf1, Wf2, bf2):
    raise NotImplementedError("write your pallas kernel here")


# speedup vs baseline: 6.6150x; 6.6150x over previous
import functools
import jax
import jax.numpy as jnp
from jax import lax
from jax.experimental import pallas as pl
from jax.experimental.pallas import tpu as pltpu
from jax.experimental.pallas import tpu_sc as plsc

N = 10000
E = 320000
D = 128
H = 256
ED = 16
G = 100
NPG = 100
MLP_DIM = 512
C = 10
ENCI = 12

XP = 144
NC = 2
NS = 16
K = 80
NP = 10240
RPT = NP // NS

def _mesh():
    return plsc.VectorSubcoreMesh(core_axis_name="c", subcore_axis_name="s",
                                  num_cores=NC, num_subcores=NS)


def _sc_phase_a(xpad, srcv, dstv, ea, z144, z16,
                outx, oute,
                idxs, idxd, rows, earows, accx, acce, sem):
    c = lax.axis_index("c")
    s = lax.axis_index("s")
    wid = s * NC + c
    tb = s * RPT
    pltpu.sync_copy(z144.at[pl.ds(tb, RPT)],
                    accx.at[pl.ds(tb, RPT)])
    pltpu.sync_copy(z16.at[pl.ds(tb, RPT)],
                    acce.at[pl.ds(tb, RPT)])
    plsc.subcore_barrier()

    edges_per_tile = E // (NC * NS)
    nchunk = edges_per_tile // K
    ebase = wid * edges_per_tile

    def body(j, carry):
        base = ebase + j * K
        pltpu.sync_copy(srcv.at[pl.ds(base, K)], idxs)
        pltpu.sync_copy(dstv.at[pl.ds(base, K)], idxd)
        pltpu.async_copy(xpad.at[idxs], rows, sem).wait()
        pltpu.sync_copy(ea.at[pl.ds(base, K)], earows)
        pltpu.sync_copy(rows, accx.at[idxd], add=True)
        pltpu.sync_copy(earows, acce.at[idxd], add=True)
        return carry

    lax.fori_loop(0, nchunk, body, 0)
    plsc.subcore_barrier()
    pltpu.sync_copy(accx.at[pl.ds(tb, RPT)],
                    outx.at[pl.ds(c * NP + tb, RPT)])
    pltpu.sync_copy(acce.at[pl.ds(tb, RPT)],
                    oute.at[pl.ds(c * NP + tb, RPT)])


def _make_phase_a():
  return pl.kernel(
    _sc_phase_a,
    out_type=(jax.ShapeDtypeStruct((NC * NP, XP), jnp.float32),
              jax.ShapeDtypeStruct((NC * NP, ED), jnp.float32)),
    mesh=_mesh(),
    scratch_types=[
        pltpu.VMEM((K,), jnp.int32),
        pltpu.VMEM((K,), jnp.int32),
        pltpu.VMEM((K, XP), jnp.float32),
        pltpu.VMEM((K, ED), jnp.float32),
        pltpu.VMEM_SHARED((NP, XP), jnp.float32),
        pltpu.VMEM_SHARED((NP, ED), jnp.float32),
        pltpu.SemaphoreType.DMA,
    ],
    compiler_params=pltpu.CompilerParams(use_tc_tiling_on_sc=False),
  )


def _sc_phase_b(h0split, srcv, dstv, z128,
                outh,
                idxs, idxd, rows, acch, sem):
    c = lax.axis_index("c")
    s = lax.axis_index("s")
    tb = s * RPT
    pltpu.sync_copy(z128.at[pl.ds(tb, RPT)],
                    acch.at[pl.ds(tb, RPT)])
    plsc.subcore_barrier()

    edges_per_tile = E // NS
    nchunk = edges_per_tile // K
    ebase = s * edges_per_tile
    roff = c * NP

    def body(j, carry):
        base = ebase + j * K
        pltpu.sync_copy(srcv.at[pl.ds(base, K)], idxs)
        for i in range(K // 16):
            sl = pl.ds(i * 16, 16)
            idxs[sl] = idxs[sl] + roff
        pltpu.sync_copy(dstv.at[pl.ds(base, K)], idxd)
        pltpu.async_copy(h0split.at[idxs], rows, sem).wait()
        pltpu.sync_copy(rows, acch.at[idxd], add=True)
        return carry

    lax.fori_loop(0, nchunk, body, 0)
    plsc.subcore_barrier()
    pltpu.sync_copy(acch.at[pl.ds(tb, RPT)],
                    outh.at[pl.ds(c * NP + tb, RPT)])


def _make_phase_b():
  return pl.kernel(
    _sc_phase_b,
    out_type=jax.ShapeDtypeStruct((NC * NP, D), jnp.float32),
    mesh=_mesh(),
    scratch_types=[
        pltpu.VMEM((K,), jnp.int32),
        pltpu.VMEM((K,), jnp.int32),
        pltpu.VMEM((K, D), jnp.float32),
        pltpu.VMEM_SHARED((NP, D), jnp.float32),
        pltpu.SemaphoreType.DMA,
    ],
    compiler_params=pltpu.CompilerParams(use_tc_tiling_on_sc=False),
  )


RB = 1000
NG = N // RB


def _k1a_body(x_ref, aggx_ref, agge_ref, w0d_ref, w0s_ref, w0e_ref, b0_ref,
              mp_ref, aux_ref, st_ref):
    x = x_ref[...]
    aggx = aggx_ref[0] [:, 0:D] + aggx_ref[1][:, 0:D] + x
    cnt = aggx_ref[0][:, D:D + 1] + aggx_ref[1][:, D:D + 1] + 1.0
    ea = agge_ref[0] + agge_ref[1]
    w0e = w0e_ref[...]
    u = jnp.dot(x, w0d_ref[...], preferred_element_type=jnp.float32) + b0_ref[...]
    pre = (cnt * u
           + jnp.dot(aggx, w0s_ref[...], preferred_element_type=jnp.float32)
           + jnp.dot(ea, w0e, preferred_element_type=jnp.float32)
           + jnp.sum(w0e, axis=0, keepdims=True))
    mp = jnp.maximum(pre, 0.0)
    mp_ref[...] = mp
    aux_ref[0] = jnp.broadcast_to(cnt, (RB, ED))
    aux_ref[1] = ea

    @pl.when(pl.program_id(0) == 0)
    def _():
        st_ref[...] = jnp.zeros_like(st_ref)
    st_ref[0:1] += jnp.sum(mp, axis=0, keepdims=True)
    st_ref[1:2] += jnp.sum(mp * mp, axis=0, keepdims=True)


_k1a = pl.pallas_call(
    _k1a_body,
    grid=(NG,),
    in_specs=[
        pl.BlockSpec((RB, D), lambda i: (i, 0)),
        pl.BlockSpec((2, RB, XP), lambda i: (0, i, 0)),
        pl.BlockSpec((2, RB, ED), lambda i: (0, i, 0)),
        pl.BlockSpec((D, H), lambda i: (0, 0)),
        pl.BlockSpec((D, H), lambda i: (0, 0)),
        pl.BlockSpec((ED, H), lambda i: (0, 0)),
        pl.BlockSpec((1, H), lambda i: (0, 0)),
    ],
    out_specs=[
        pl.BlockSpec((RB, H), lambda i: (i, 0)),
        pl.BlockSpec((2, RB, ED), lambda i: (0, i, 0)),
        pl.BlockSpec((2, H), lambda i: (0, 0)),
    ],
    out_shape=(jax.ShapeDtypeStruct((N, H), jnp.float32),
               jax.ShapeDtypeStruct((2, N, ED), jnp.float32),
               jax.ShapeDtypeStruct((2, H), jnp.float32)),
)


def _k1b_body(mp_ref, st_ref, g_ref, be_ref, h0_ref):
    mp = mp_ref[...]
    mu = st_ref[0:1] * (1.0 / N)
    var = st_ref[1:2] * (1.0 / N) - mu * mu
    h = jnp.maximum((mp - mu) * lax.rsqrt(var + 1e-5) * g_ref[...] + be_ref[...],
                    0.0)
    h0_ref[0] = h[:, 0:D]
    h0_ref[1] = h[:, D:2 * D]


_k1b = pl.pallas_call(
    _k1b_body,
    grid=(NG,),
    in_specs=[
        pl.BlockSpec((RB, H), lambda i: (i, 0)),
        pl.BlockSpec((2, H), lambda i: (0, 0)),
        pl.BlockSpec((1, H), lambda i: (0, 0)),
        pl.BlockSpec((1, H), lambda i: (0, 0)),
    ],
    out_specs=pl.BlockSpec((2, RB, D), lambda i: (0, i, 0)),
    out_shape=jax.ShapeDtypeStruct((2, NP, D), jnp.float32),
)


def _k2a_body(h0_ref, aggh_ref, aux_ref, w1d_ref, w1s_ref, w1e_ref, b1_ref,
              mp_ref, st_ref):
    h0 = jnp.concatenate([h0_ref[0], h0_ref[1]], axis=1)
    aggh = jnp.concatenate([aggh_ref[0], aggh_ref[1]], axis=1) + h0
    cnt = aux_ref[0][:, 0:1]
    ea = aux_ref[1][...]
    w1e = w1e_ref[...]
    u = jnp.dot(h0, w1d_ref[...], preferred_element_type=jnp.float32) + b1_ref[...]
    pre = (cnt * u
           + jnp.dot(aggh, w1s_ref[...], preferred_element_type=jnp.float32)
           + jnp.dot(ea, w1e, preferred_element_type=jnp.float32)
           + jnp.sum(w1e, axis=0, keepdims=True))
    mp = jnp.maximum(pre, 0.0)
    mp_ref[...] = mp

    @pl.when(pl.program_id(0) == 0)
    def _():
        st_ref[...] = jnp.zeros_like(st_ref)
    st_ref[0:1] += jnp.sum(mp, axis=0, keepdims=True)
    st_ref[1:2] += jnp.sum(mp * mp, axis=0, keepdims=True)


_k2a = pl.pallas_call(
    _k2a_body,
    grid=(NG,),
    in_specs=[
        pl.BlockSpec((2, RB, D), lambda i: (0, i, 0)),
        pl.BlockSpec((2, RB, D), lambda i: (0, i, 0)),
        pl.BlockSpec((2, RB, ED), lambda i: (0, i, 0)),
        pl.BlockSpec((H, H), lambda i: (0, 0)),
        pl.BlockSpec((H, H), lambda i: (0, 0)),
        pl.BlockSpec((ED, H), lambda i: (0, 0)),
        pl.BlockSpec((1, H), lambda i: (0, 0)),
    ],
    out_specs=[
        pl.BlockSpec((RB, H), lambda i: (i, 0)),
        pl.BlockSpec((2, H), lambda i: (0, 0)),
    ],
    out_shape=(jax.ShapeDtypeStruct((N, H), jnp.float32),
               jax.ShapeDtypeStruct((2, H), jnp.float32)),
)


GB = G // NG


def _k2b_body(mp_ref, h0_ref, st_ref, g_ref, be_ref,
              pooled_ref, ne0_ref, ne1_ref):
    mp = mp_ref[...]
    mu = st_ref[0:1] * (1.0 / N)
    var = st_ref[1:2] * (1.0 / N) - mu * mu
    h1 = jnp.maximum((mp - mu) * lax.rsqrt(var + 1e-5) * g_ref[...] + be_ref[...],
                     0.0)
    h0 = jnp.concatenate([h0_ref[0], h0_ref[1]], axis=1)
    col = lax.broadcasted_iota(jnp.int32, (GB, RB), 1)
    row = lax.broadcasted_iota(jnp.int32, (GB, RB), 0)
    seg = jnp.where((col >= row * NPG) & (col < row * NPG + NPG), 1.0, 0.0)
    sel = jnp.where(col == row * NPG, 1.0, 0.0)
    pooled_ref[0] = jnp.dot(seg, h1, preferred_element_type=jnp.float32)
    ne0_ref[0] = jnp.dot(sel, h0, preferred_element_type=jnp.float32)
    ne1_ref[0] = jnp.dot(sel, h1, preferred_element_type=jnp.float32)


_k2b = pl.pallas_call(
    _k2b_body,
    grid=(NG,),
    in_specs=[
        pl.BlockSpec((RB, H), lambda i: (i, 0)),
        pl.BlockSpec((2, RB, D), lambda i: (0, i, 0)),
        pl.BlockSpec((2, H), lambda i: (0, 0)),
        pl.BlockSpec((1, H), lambda i: (0, 0)),
        pl.BlockSpec((1, H), lambda i: (0, 0)),
    ],
    out_specs=[
        pl.BlockSpec((1, GB, H), lambda i: (i, 0, 0)),
        pl.BlockSpec((1, GB, H), lambda i: (i, 0, 0)),
        pl.BlockSpec((1, GB, H), lambda i: (i, 0, 0)),
    ],
    out_shape=(jax.ShapeDtypeStruct((NG, GB, H), jnp.float32),
               jax.ShapeDtypeStruct((NG, GB, H), jnp.float32),
               jax.ShapeDtypeStruct((NG, GB, H), jnp.float32)),
)


def _k2c_body(pooled_ref, ne0_ref, ne1_ref, wf1_ref, bf1_ref, wf2_ref, bf2_ref,
              out_ref):
    z = jnp.concatenate([pooled_ref[...], ne0_ref[...], ne1_ref[...]], axis=1)
    z = jnp.maximum(jnp.dot(z, wf1_ref[...], preferred_element_type=jnp.float32)
                    + bf1_ref[...], 0.0)
    out_ref[...] = (jnp.dot(z, wf2_ref[...], preferred_element_type=jnp.float32)
                    + bf2_ref[...])


_k2c = pl.pallas_call(
    _k2c_body,
    out_shape=jax.ShapeDtypeStruct((G, C), jnp.float32),
)


@jax.jit
def kernel(x, edge_index, edge_attr, batch, mask,
           W0, b0, g0, be0, W1, b1, g1, be1, Wf1, bf1, Wf2, bf2):
    src = edge_index[0]
    dst = edge_index[1]
    ones_col = jnp.ones((N, 1), jnp.float32)
    pad = jnp.zeros((N, XP - D - 1), jnp.float32)
    xpad = jnp.concatenate([x, ones_col, pad], axis=1)
    z144 = jnp.zeros((NP, XP), jnp.float32)
    z128 = jnp.zeros((NP, D), jnp.float32)
    z16 = jnp.zeros((NP, ED), jnp.float32)

    aggx, agge = _make_phase_a()(xpad, src, dst, edge_attr, z144, z16)
    aggx3 = aggx.reshape(2, NP, XP)
    agge3 = agge.reshape(2, NP, ED)

    mp0, aux, st0 = _k1a(x, aggx3, agge3, W0[0:D], W0[D:2 * D], W0[2 * D:],
                         b0.reshape(1, H))
    h0split3 = _k1b(mp0, st0, g0.reshape(1, H), be0.reshape(1, H))

    aggh = _make_phase_b()(h0split3.reshape(2 * NP, D), src, dst, z128)

    mp1, st1 = _k2a(h0split3, aggh.reshape(2, NP, D), aux,
                    W1[0:H], W1[H:2 * H], W1[2 * H:], b1.reshape(1, H))
    pooled, ne0, ne1 = _k2b(mp1, h0split3, st1,
                            g1.reshape(1, H), be1.reshape(1, H))
    return _k2c(pooled.reshape(G, H), ne0.reshape(G, H), ne1.reshape(G, H),
                Wf1, bf1.reshape(1, MLP_DIM), Wf2, bf2.reshape(1, C))


# speedup vs baseline: 11.8548x; 1.7921x over previous
import functools
import jax
import jax.numpy as jnp
from jax import lax
from jax.experimental import pallas as pl
from jax.experimental.pallas import tpu as pltpu
from jax.experimental.pallas import tpu_sc as plsc

N = 10000
E = 320000
D = 128
H = 256
ED = 16
G = 100
NPG = 100
MLP_DIM = 512
C = 10
ENCI = 12

XP = 144
NC = 2
NS = 16
K = 80
NP = 10240
RPT = NP // NS

def _mesh():
    return plsc.VectorSubcoreMesh(core_axis_name="c", subcore_axis_name="s",
                                  num_cores=NC, num_subcores=NS)


def _sc_phase_a(xpad, srcr, dstr, ea, z144, z16,
                outx, oute,
                idxsA, idxsB, idxdA, idxdB, rowsA, rowsB, eaA, eaB,
                accx, acce,
                ssA, ssB, sdA, sdB, sgA, sgB, seA, seB):
    c = lax.axis_index("c")
    s = lax.axis_index("s")
    wid = s * NC + c
    tb = s * RPT
    pltpu.sync_copy(z144.at[pl.ds(tb, RPT)], accx.at[pl.ds(tb, RPT)])
    pltpu.sync_copy(z16.at[pl.ds(tb, RPT)], acce.at[pl.ds(tb, RPT)])

    nchunk = E // (NC * NS) // K
    crow = wid * nchunk

    def issue_idx(j, idxs, idxd, ss, sd):
        pltpu.async_copy(srcr.at[pl.ds((crow + j) * K, K)], idxs, ss)
        pltpu.async_copy(dstr.at[pl.ds((crow + j) * K, K)], idxd, sd)

    def wait_idx(j, idxs, idxd, ss, sd):
        pltpu.make_async_copy(srcr.at[pl.ds((crow + j) * K, K)], idxs, ss).wait()
        pltpu.make_async_copy(dstr.at[pl.ds((crow + j) * K, K)], idxd, sd).wait()

    def gather(j, idxs, rows, earows, sg, se):
        pltpu.async_copy(xpad.at[idxs], rows, sg)
        pltpu.async_copy(ea.at[pl.ds((crow + j) * K, K)], earows, se)

    def scatter(j, idxs, idxd, rows, earows, sg, se):
        pltpu.make_async_copy(xpad.at[idxs], rows, sg).wait()
        pltpu.sync_copy(rows, accx.at[idxd], add=True)
        pltpu.make_async_copy(ea.at[pl.ds((crow + j) * K, K)], earows, se).wait()
        pltpu.sync_copy(earows, acce.at[idxd], add=True)

    plsc.subcore_barrier()
    issue_idx(0, idxsA, idxdA, ssA, sdA)
    issue_idx(1, idxsB, idxdB, ssB, sdB)

    def body(i, carry):
        a = 2 * i
        b = a + 1
        wait_idx(a, idxsA, idxdA, ssA, sdA)
        gather(a, idxsA, rowsA, eaA, sgA, seA)

        @pl.when(b < nchunk)
        def _():
            wait_idx(b, idxsB, idxdB, ssB, sdB)
            gather(b, idxsB, rowsB, eaB, sgB, seB)
        scatter(a, idxsA, idxdA, rowsA, eaA, sgA, seA)

        @pl.when(a + 2 < nchunk)
        def _():
            issue_idx(a + 2, idxsA, idxdA, ssA, sdA)

        @pl.when(b < nchunk)
        def _():
            scatter(b, idxsB, idxdB, rowsB, eaB, sgB, seB)

            @pl.when(b + 2 < nchunk)
            def _():
                issue_idx(b + 2, idxsB, idxdB, ssB, sdB)
        return carry

    lax.fori_loop(0, (nchunk + 1) // 2, body, 0)
    plsc.subcore_barrier()
    pltpu.sync_copy(accx.at[pl.ds(tb, RPT)], outx.at[pl.ds(c * NP + tb, RPT)])
    pltpu.sync_copy(acce.at[pl.ds(tb, RPT)], oute.at[pl.ds(c * NP + tb, RPT)])


def _make_phase_a():
  return pl.kernel(
    _sc_phase_a,
    out_type=(jax.ShapeDtypeStruct((NC * NP, XP), jnp.float32),
              jax.ShapeDtypeStruct((NC * NP, ED), jnp.float32)),
    mesh=_mesh(),
    scratch_types=[
        pltpu.VMEM((K,), jnp.int32),
        pltpu.VMEM((K,), jnp.int32),
        pltpu.VMEM((K,), jnp.int32),
        pltpu.VMEM((K,), jnp.int32),
        pltpu.VMEM((K, XP), jnp.float32),
        pltpu.VMEM((K, XP), jnp.float32),
        pltpu.VMEM((K, ED), jnp.float32),
        pltpu.VMEM((K, ED), jnp.float32),
        pltpu.VMEM_SHARED((NP, XP), jnp.float32),
        pltpu.VMEM_SHARED((NP, ED), jnp.float32),
        pltpu.SemaphoreType.DMA,
        pltpu.SemaphoreType.DMA,
        pltpu.SemaphoreType.DMA,
        pltpu.SemaphoreType.DMA,
        pltpu.SemaphoreType.DMA,
        pltpu.SemaphoreType.DMA,
        pltpu.SemaphoreType.DMA,
        pltpu.SemaphoreType.DMA,
    ],
    compiler_params=pltpu.CompilerParams(use_tc_tiling_on_sc=False),
  )


def _sc_phase_b(h0split, src2r, dstr, z128,
                outh,
                idxsA, idxsB, idxdA, idxdB, rowsA, rowsB, acch,
                ssA, ssB, sdA, sdB, sgA, sgB):
    c = lax.axis_index("c")
    s = lax.axis_index("s")
    tb = s * RPT
    pltpu.sync_copy(z128.at[pl.ds(tb, RPT)], acch.at[pl.ds(tb, RPT)])

    nchunk = E // NS // K
    sbase = (c * (E // K) + s * nchunk) * K
    dbase = s * nchunk * K

    def issue_idx(j, idxs, idxd, ss, sd):
        pltpu.async_copy(src2r.at[pl.ds(sbase + j * K, K)], idxs, ss)
        pltpu.async_copy(dstr.at[pl.ds(dbase + j * K, K)], idxd, sd)

    def wait_idx(j, idxs, idxd, ss, sd):
        pltpu.make_async_copy(src2r.at[pl.ds(sbase + j * K, K)], idxs, ss).wait()
        pltpu.make_async_copy(dstr.at[pl.ds(dbase + j * K, K)], idxd, sd).wait()

    def gather(idxs, rows, sg):
        pltpu.async_copy(h0split.at[idxs], rows, sg)

    def scatter(idxs, idxd, rows, sg):
        pltpu.make_async_copy(h0split.at[idxs], rows, sg).wait()
        pltpu.sync_copy(rows, acch.at[idxd], add=True)

    plsc.subcore_barrier()
    issue_idx(0, idxsA, idxdA, ssA, sdA)
    issue_idx(1, idxsB, idxdB, ssB, sdB)

    def body(i, carry):
        a = 2 * i
        b = a + 1
        wait_idx(a, idxsA, idxdA, ssA, sdA)
        gather(idxsA, rowsA, sgA)
        wait_idx(b, idxsB, idxdB, ssB, sdB)
        gather(idxsB, rowsB, sgB)
        scatter(idxsA, idxdA, rowsA, sgA)

        @pl.when(a + 2 < nchunk)
        def _():
            issue_idx(a + 2, idxsA, idxdA, ssA, sdA)
        scatter(idxsB, idxdB, rowsB, sgB)

        @pl.when(b + 2 < nchunk)
        def _():
            issue_idx(b + 2, idxsB, idxdB, ssB, sdB)
        return carry

    lax.fori_loop(0, nchunk // 2, body, 0)
    plsc.subcore_barrier()
    pltpu.sync_copy(acch.at[pl.ds(tb, RPT)], outh.at[pl.ds(c * NP + tb, RPT)])


def _make_phase_b():
  return pl.kernel(
    _sc_phase_b,
    out_type=jax.ShapeDtypeStruct((NC * NP, D), jnp.float32),
    mesh=_mesh(),
    scratch_types=[
        pltpu.VMEM((K,), jnp.int32),
        pltpu.VMEM((K,), jnp.int32),
        pltpu.VMEM((K,), jnp.int32),
        pltpu.VMEM((K,), jnp.int32),
        pltpu.VMEM((K, D), jnp.float32),
        pltpu.VMEM((K, D), jnp.float32),
        pltpu.VMEM_SHARED((NP, D), jnp.float32),
        pltpu.SemaphoreType.DMA,
        pltpu.SemaphoreType.DMA,
        pltpu.SemaphoreType.DMA,
        pltpu.SemaphoreType.DMA,
        pltpu.SemaphoreType.DMA,
        pltpu.SemaphoreType.DMA,
    ],
    compiler_params=pltpu.CompilerParams(use_tc_tiling_on_sc=False),
  )


RB = 1000
NG = N // RB


def _k1a_body(x_ref, aggx_ref, agge_ref, w0d_ref, w0s_ref, w0e_ref, b0_ref,
              mp_ref, aux_ref, st_ref):
    x = x_ref[...]
    aggx = aggx_ref[0] [:, 0:D] + aggx_ref[1][:, 0:D] + x
    cnt = aggx_ref[0][:, D:D + 1] + aggx_ref[1][:, D:D + 1] + 1.0
    ea = agge_ref[0] + agge_ref[1]
    w0e = w0e_ref[...]
    u = jnp.dot(x, w0d_ref[...], preferred_element_type=jnp.float32) + b0_ref[...]
    pre = (cnt * u
           + jnp.dot(aggx, w0s_ref[...], preferred_element_type=jnp.float32)
           + jnp.dot(ea, w0e, preferred_element_type=jnp.float32)
           + jnp.sum(w0e, axis=0, keepdims=True))
    mp = jnp.maximum(pre, 0.0)
    mp_ref[...] = mp
    aux_ref[0] = jnp.broadcast_to(cnt, (RB, ED))
    aux_ref[1] = ea

    @pl.when(pl.program_id(0) == 0)
    def _():
        st_ref[...] = jnp.zeros_like(st_ref)
    st_ref[0:1] += jnp.sum(mp, axis=0, keepdims=True)
    st_ref[1:2] += jnp.sum(mp * mp, axis=0, keepdims=True)


_k1a = pl.pallas_call(
    _k1a_body,
    grid=(NG,),
    in_specs=[
        pl.BlockSpec((RB, D), lambda i: (i, 0)),
        pl.BlockSpec((2, RB, XP), lambda i: (0, i, 0)),
        pl.BlockSpec((2, RB, ED), lambda i: (0, i, 0)),
        pl.BlockSpec((D, H), lambda i: (0, 0)),
        pl.BlockSpec((D, H), lambda i: (0, 0)),
        pl.BlockSpec((ED, H), lambda i: (0, 0)),
        pl.BlockSpec((1, H), lambda i: (0, 0)),
    ],
    out_specs=[
        pl.BlockSpec((RB, H), lambda i: (i, 0)),
        pl.BlockSpec((2, RB, ED), lambda i: (0, i, 0)),
        pl.BlockSpec((2, H), lambda i: (0, 0)),
    ],
    out_shape=(jax.ShapeDtypeStruct((N, H), jnp.float32),
               jax.ShapeDtypeStruct((2, N, ED), jnp.float32),
               jax.ShapeDtypeStruct((2, H), jnp.float32)),
)


def _k1b_body(mp_ref, st_ref, g_ref, be_ref, h0_ref):
    mp = mp_ref[...]
    mu = st_ref[0:1] * (1.0 / N)
    var = st_ref[1:2] * (1.0 / N) - mu * mu
    h = jnp.maximum((mp - mu) * lax.rsqrt(var + 1e-5) * g_ref[...] + be_ref[...],
                    0.0)
    h0_ref[0] = h[:, 0:D]
    h0_ref[1] = h[:, D:2 * D]


_k1b = pl.pallas_call(
    _k1b_body,
    grid=(NG,),
    in_specs=[
        pl.BlockSpec((RB, H), lambda i: (i, 0)),
        pl.BlockSpec((2, H), lambda i: (0, 0)),
        pl.BlockSpec((1, H), lambda i: (0, 0)),
        pl.BlockSpec((1, H), lambda i: (0, 0)),
    ],
    out_specs=pl.BlockSpec((2, RB, D), lambda i: (0, i, 0)),
    out_shape=jax.ShapeDtypeStruct((2, NP, D), jnp.float32),
)


def _k2a_body(h0_ref, aggh_ref, aux_ref, w1d_ref, w1s_ref, w1e_ref, b1_ref,
              mp_ref, st_ref):
    h0 = jnp.concatenate([h0_ref[0], h0_ref[1]], axis=1)
    aggh = jnp.concatenate([aggh_ref[0], aggh_ref[1]], axis=1) + h0
    cnt = aux_ref[0][:, 0:1]
    ea = aux_ref[1][...]
    w1e = w1e_ref[...]
    u = jnp.dot(h0, w1d_ref[...], preferred_element_type=jnp.float32) + b1_ref[...]
    pre = (cnt * u
           + jnp.dot(aggh, w1s_ref[...], preferred_element_type=jnp.float32)
           + jnp.dot(ea, w1e, preferred_element_type=jnp.float32)
           + jnp.sum(w1e, axis=0, keepdims=True))
    mp = jnp.maximum(pre, 0.0)
    mp_ref[...] = mp

    @pl.when(pl.program_id(0) == 0)
    def _():
        st_ref[...] = jnp.zeros_like(st_ref)
    st_ref[0:1] += jnp.sum(mp, axis=0, keepdims=True)
    st_ref[1:2] += jnp.sum(mp * mp, axis=0, keepdims=True)


_k2a = pl.pallas_call(
    _k2a_body,
    grid=(NG,),
    in_specs=[
        pl.BlockSpec((2, RB, D), lambda i: (0, i, 0)),
        pl.BlockSpec((2, RB, D), lambda i: (0, i, 0)),
        pl.BlockSpec((2, RB, ED), lambda i: (0, i, 0)),
        pl.BlockSpec((H, H), lambda i: (0, 0)),
        pl.BlockSpec((H, H), lambda i: (0, 0)),
        pl.BlockSpec((ED, H), lambda i: (0, 0)),
        pl.BlockSpec((1, H), lambda i: (0, 0)),
    ],
    out_specs=[
        pl.BlockSpec((RB, H), lambda i: (i, 0)),
        pl.BlockSpec((2, H), lambda i: (0, 0)),
    ],
    out_shape=(jax.ShapeDtypeStruct((N, H), jnp.float32),
               jax.ShapeDtypeStruct((2, H), jnp.float32)),
)


GB = G // NG


def _k2b_body(mp_ref, h0_ref, st_ref, g_ref, be_ref,
              pooled_ref, ne0_ref, ne1_ref):
    mp = mp_ref[...]
    mu = st_ref[0:1] * (1.0 / N)
    var = st_ref[1:2] * (1.0 / N) - mu * mu
    h1 = jnp.maximum((mp - mu) * lax.rsqrt(var + 1e-5) * g_ref[...] + be_ref[...],
                     0.0)
    h0 = jnp.concatenate([h0_ref[0], h0_ref[1]], axis=1)
    col = lax.broadcasted_iota(jnp.int32, (GB, RB), 1)
    row = lax.broadcasted_iota(jnp.int32, (GB, RB), 0)
    seg = jnp.where((col >= row * NPG) & (col < row * NPG + NPG), 1.0, 0.0)
    sel = jnp.where(col == row * NPG, 1.0, 0.0)
    pooled_ref[0] = jnp.dot(seg, h1, preferred_element_type=jnp.float32)
    ne0_ref[0] = jnp.dot(sel, h0, preferred_element_type=jnp.float32)
    ne1_ref[0] = jnp.dot(sel, h1, preferred_element_type=jnp.float32)


_k2b = pl.pallas_call(
    _k2b_body,
    grid=(NG,),
    in_specs=[
        pl.BlockSpec((RB, H), lambda i: (i, 0)),
        pl.BlockSpec((2, RB, D), lambda i: (0, i, 0)),
        pl.BlockSpec((2, H), lambda i: (0, 0)),
        pl.BlockSpec((1, H), lambda i: (0, 0)),
        pl.BlockSpec((1, H), lambda i: (0, 0)),
    ],
    out_specs=[
        pl.BlockSpec((1, GB, H), lambda i: (i, 0, 0)),
        pl.BlockSpec((1, GB, H), lambda i: (i, 0, 0)),
        pl.BlockSpec((1, GB, H), lambda i: (i, 0, 0)),
    ],
    out_shape=(jax.ShapeDtypeStruct((NG, GB, H), jnp.float32),
               jax.ShapeDtypeStruct((NG, GB, H), jnp.float32),
               jax.ShapeDtypeStruct((NG, GB, H), jnp.float32)),
)


def _k2c_body(pooled_ref, ne0_ref, ne1_ref, wf1_ref, bf1_ref, wf2_ref, bf2_ref,
              out_ref):
    z = jnp.concatenate([pooled_ref[...], ne0_ref[...], ne1_ref[...]], axis=1)
    z = jnp.maximum(jnp.dot(z, wf1_ref[...], preferred_element_type=jnp.float32)
                    + bf1_ref[...], 0.0)
    out_ref[...] = (jnp.dot(z, wf2_ref[...], preferred_element_type=jnp.float32)
                    + bf2_ref[...])


_k2c = pl.pallas_call(
    _k2c_body,
    out_shape=jax.ShapeDtypeStruct((G, C), jnp.float32),
)


@jax.jit
def kernel(x, edge_index, edge_attr, batch, mask,
           W0, b0, g0, be0, W1, b1, g1, be1, Wf1, bf1, Wf2, bf2):
    src = edge_index[0]
    dst = edge_index[1]
    ones_col = jnp.ones((N, 1), jnp.float32)
    pad = jnp.zeros((N, XP - D - 1), jnp.float32)
    xpad = jnp.concatenate([x, ones_col, pad], axis=1)
    z144 = jnp.zeros((NP, XP), jnp.float32)
    z128 = jnp.zeros((NP, D), jnp.float32)
    z16 = jnp.zeros((NP, ED), jnp.float32)

    src2 = jnp.concatenate([src, src + NP], axis=0)
    aggx, agge = _make_phase_a()(xpad, src, dst, edge_attr, z144, z16)
    aggx3 = aggx.reshape(2, NP, XP)
    agge3 = agge.reshape(2, NP, ED)

    mp0, aux, st0 = _k1a(x, aggx3, agge3, W0[0:D], W0[D:2 * D], W0[2 * D:],
                         b0.reshape(1, H))
    h0split3 = _k1b(mp0, st0, g0.reshape(1, H), be0.reshape(1, H))

    aggh = _make_phase_b()(h0split3.reshape(2 * NP, D), src2, dst, z128)

    mp1, st1 = _k2a(h0split3, aggh.reshape(2, NP, D), aux,
                    W1[0:H], W1[H:2 * H], W1[2 * H:], b1.reshape(1, H))
    pooled, ne0, ne1 = _k2b(mp1, h0split3, st1,
                            g1.reshape(1, H), be1.reshape(1, H))
    return _k2c(pooled.reshape(G, H), ne0.reshape(G, H), ne1.reshape(G, H),
                Wf1, bf1.reshape(1, MLP_DIM), Wf2, bf2.reshape(1, C))
